# Initial kernel scaffold; baseline (speedup 1.0000x reference)
import jax
import jax.numpy as jnp
from jax.experimental import pallas as pl


def kernel(text, table, W, b):
    raise NotImplementedError("write your pallas kernel here")


# speedup vs baseline: 18.3078x; 18.3078x over previous
import functools

import jax
import jax.numpy as jnp
from jax import lax
from jax.experimental import pallas as pl
from jax.experimental.pallas import tpu as pltpu
from jax.experimental.pallas import tpu_sc as plsc

VOCAB = 100000
EMBED = 64
OUT = 2
SEQ = 200
BATCH = 4096

NC = 2
NS = 16
L = 16
NW = NC * NS
BPW = BATCH // NW
G = BPW // L

TBLK = 2048


def _project_body(table_ref, w_ref, o_ref):
    t = table_ref[...]
    w = w_ref[...]
    inv_s = jnp.float32(1.0 / SEQ)
    p0 = jnp.sum(t * w[0:1, :], axis=1) * inv_s
    p1 = jnp.sum(t * w[1:2, :], axis=1) * inv_s
    u0 = lax.bitcast_convert_type(p0.astype(jnp.bfloat16), jnp.uint16)
    u1 = lax.bitcast_convert_type(p1.astype(jnp.bfloat16), jnp.uint16)
    word = u0.astype(jnp.int32) | (u1.astype(jnp.int32) << 16)
    o_ref[...] = word


def _project_table(table, W):
    nblk = pl.cdiv(VOCAB, TBLK)
    return pl.pallas_call(
        _project_body,
        grid=(nblk,),
        in_specs=[
            pl.BlockSpec((TBLK, EMBED), lambda i: (i, 0)),
            pl.BlockSpec((OUT, EMBED), lambda i: (0, 0)),
        ],
        out_specs=pl.BlockSpec((TBLK,), lambda i: (i,)),
        out_shape=jax.ShapeDtypeStruct((VOCAB,), jnp.int32),
    )(table, W)


def _sc_body(ptable_hbm, text_hbm, b0_hbm, b1_hbm, out_hbm,
             table_v, text_v, out_v, b0_v, b1_v, sem_t, sem_x):
    wid = lax.axis_index("s") * NC + lax.axis_index("c")
    base = wid * BPW

    cp_t = pltpu.make_async_copy(ptable_hbm, table_v, sem_t)
    cp_t.start()
    cp_x = pltpu.make_async_copy(
        text_hbm.at[:, pl.ds(base, BPW)], text_v, sem_x)
    cp_x.start()
    pltpu.sync_copy(b0_hbm, b0_v)
    pltpu.sync_copy(b1_hbm, b1_v)
    cp_t.wait()
    cp_x.wait()

    bias0 = b0_v[...]
    bias1 = b1_v[...]
    zero = jnp.zeros((L,), jnp.float32)

    for g in range(G):
        def body(s, carry, g=g):
            a0, a1 = carry
            idx = text_v[s, pl.ds(g * L, L)]
            word = plsc.load_gather(table_v, [idx])
            bf = plsc.bitcast(word, jnp.bfloat16)
            x0, x1 = plsc.unpack(bf, format=plsc.PackFormat.INTERLEAVED)
            return a0 + x0, a1 + x1

        a0, a1 = lax.fori_loop(0, SEQ, body, (zero, zero))
        flat0 = (lax.iota(jnp.int32, L) + jnp.int32(g * L)) * OUT
        plsc.store_scatter(out_v, [flat0], a0 + bias0)
        plsc.store_scatter(out_v, [flat0 + 1], a1 + bias1)

    pltpu.sync_copy(out_v, out_hbm.at[pl.ds(base * OUT, BPW * OUT)])


@functools.cache
def _sc_reduce():
    return pl.kernel(
        _sc_body,
        out_type=jax.ShapeDtypeStruct((BATCH * OUT,), jnp.float32),
        mesh=plsc.VectorSubcoreMesh(core_axis_name="c", subcore_axis_name="s",
                                    num_cores=NC, num_subcores=NS),
        compiler_params=pltpu.CompilerParams(needs_layout_passes=False),
        scratch_types=[
            pltpu.VMEM((VOCAB,), jnp.int32),
            pltpu.VMEM((SEQ, BPW), jnp.int32),
            pltpu.VMEM((BPW * OUT,), jnp.float32),
            pltpu.VMEM((L,), jnp.float32),
            pltpu.VMEM((L,), jnp.float32),
            pltpu.SemaphoreType.DMA,
            pltpu.SemaphoreType.DMA,
        ],
    )


def kernel(text, table, W, b):
    ptable = _project_table(table, W)
    b0 = jnp.broadcast_to(b[0], (L,))
    b1 = jnp.broadcast_to(b[1], (L,))
    flat = _sc_reduce()(ptable, text, b0, b1)
    return flat.reshape(BATCH, OUT)


# speedup vs baseline: 56.0968x; 3.0641x over previous
import functools

import jax
import jax.numpy as jnp
from jax import lax
from jax.experimental import pallas as pl
from jax.experimental.pallas import tpu as pltpu
from jax.experimental.pallas import tpu_sc as plsc

VOCAB = 100000
EMBED = 64
OUT = 2
SEQ = 200
BATCH = 4096

NC = 2
NS = 16
L = 16
NW = NC * NS
BPW = BATCH // NW
G = BPW // L

TBLK = 4096


def _project_body(tablet_ref, w_ref, o_ref):
    t = tablet_ref[...]
    w = w_ref[...]
    inv_s = jnp.float32(1.0 / SEQ)
    p = jnp.dot(w, t, preferred_element_type=jnp.float32) * inv_s
    u = lax.bitcast_convert_type(p.astype(jnp.bfloat16), jnp.uint16)
    word = u[0, :].astype(jnp.int32) | (u[1, :].astype(jnp.int32) << 16)
    o_ref[...] = word


def _project_table(tablet, W):
    nblk = pl.cdiv(VOCAB, TBLK)
    return pl.pallas_call(
        _project_body,
        grid=(nblk,),
        in_specs=[
            pl.BlockSpec((EMBED, TBLK), lambda i: (0, i)),
            pl.BlockSpec((OUT, EMBED), lambda i: (0, 0)),
        ],
        out_specs=pl.BlockSpec((TBLK,), lambda i: (i,)),
        out_shape=jax.ShapeDtypeStruct((VOCAB,), jnp.int32),
    )(tablet, W)


def _sc_body(ptable_hbm, text_hbm, b_hbm, out_hbm,
             table_v, text_v, out_v, b_s, sem_t, sem_x):
    wid = lax.axis_index("s") * NC + lax.axis_index("c")
    base = wid * BPW

    cp_t = pltpu.make_async_copy(ptable_hbm, table_v, sem_t)
    cp_t.start()
    cp_x = pltpu.make_async_copy(
        text_hbm.at[:, pl.ds(base, BPW)], text_v, sem_x)
    cp_x.start()
    pltpu.sync_copy(b_hbm, b_s.at[pl.ds(0, OUT)])
    cp_t.wait()
    cp_x.wait()

    bv = b_s[...]
    bias0 = bv[0]
    bias1 = bv[1]
    zero = jnp.zeros((L,), jnp.float32)

    def body(s, carry):
        new = []
        for g in range(G):
            a0, a1 = carry[2 * g], carry[2 * g + 1]
            idx = text_v[s, pl.ds(g * L, L)]
            w = plsc.load_gather(table_v, [idx])
            x0, x1 = plsc.unpack(plsc.bitcast(w, jnp.bfloat16),
                                 format=plsc.PackFormat.INTERLEAVED)
            new.extend((a0 + x0, a1 + x1))
        return tuple(new)

    accs = plsc.parallel_loop(
        0, SEQ, 1, unroll=2, carry=(zero,) * (2 * G))(body)
    for g in range(G):
        out_v[0, pl.ds(g * L, L)] = accs[2 * g] + bias0
        out_v[1, pl.ds(g * L, L)] = accs[2 * g + 1] + bias1

    pltpu.sync_copy(out_v, out_hbm.at[:, pl.ds(base, BPW)])


@functools.cache
def _sc_reduce():
    return pl.kernel(
        _sc_body,
        out_type=jax.ShapeDtypeStruct((OUT, BATCH), jnp.float32),
        mesh=plsc.VectorSubcoreMesh(core_axis_name="c", subcore_axis_name="s",
                                    num_cores=NC, num_subcores=NS),
        compiler_params=pltpu.CompilerParams(needs_layout_passes=False),
        scratch_types=[
            pltpu.VMEM((VOCAB,), jnp.int32),
            pltpu.VMEM((SEQ, BPW), jnp.int32),
            pltpu.VMEM((OUT, BPW), jnp.float32),
            pltpu.VMEM((L,), jnp.float32),
            pltpu.SemaphoreType.DMA,
            pltpu.SemaphoreType.DMA,
        ],
    )


def kernel(text, table, W, b):
    ptable = _project_table(table.T, W)
    out_t = _sc_reduce()(ptable, text, b)
    return out_t.T


# speedup vs baseline: 64.0139x; 1.1411x over previous
import functools

import jax
import jax.numpy as jnp
from jax import lax
from jax.experimental import pallas as pl
from jax.experimental.pallas import tpu as pltpu
from jax.experimental.pallas import tpu_sc as plsc

VOCAB = 100000
EMBED = 64
OUT = 2
SEQ = 200
BATCH = 4096

NC = 2
NS = 16
L = 16
NW = NC * NS
BPW = BATCH // NW
G = BPW // L

TBLK = 8192


def _project_body(tablet_ref, w_ref, o_ref):
    t = tablet_ref[...]
    w = w_ref[...]
    inv_s = jnp.float32(1.0 / SEQ)
    p = jnp.dot(w, t, preferred_element_type=jnp.float32) * inv_s
    u = lax.bitcast_convert_type(p.astype(jnp.bfloat16), jnp.uint16)
    word = u[0, :].astype(jnp.int32) | (u[1, :].astype(jnp.int32) << 16)
    o_ref[...] = word


def _project_table(tablet, W):
    nblk = pl.cdiv(VOCAB, TBLK)
    return pl.pallas_call(
        _project_body,
        grid=(nblk,),
        in_specs=[
            pl.BlockSpec((EMBED, TBLK), lambda i: (0, i)),
            pl.BlockSpec((OUT, EMBED), lambda i: (0, 0)),
        ],
        out_specs=pl.BlockSpec((TBLK,), lambda i: (i,)),
        out_shape=jax.ShapeDtypeStruct((VOCAB,), jnp.int32),
    )(tablet, W)


def _sc_body(ptable_hbm, text_hbm, b_hbm, out_hbm,
             table_v, text_v, out_v, b_s, sem_t, sem_x):
    wid = lax.axis_index("s") * NC + lax.axis_index("c")
    base = wid * BPW

    cp_t = pltpu.make_async_copy(ptable_hbm, table_v, sem_t)
    cp_t.start()
    cp_x = pltpu.make_async_copy(
        text_hbm.at[:, pl.ds(base, BPW)], text_v, sem_x)
    cp_x.start()
    pltpu.sync_copy(b_hbm, b_s.at[pl.ds(0, OUT)])
    cp_t.wait()
    cp_x.wait()

    bv = b_s[...]
    bias0 = bv[0]
    bias1 = bv[1]
    zero = jnp.zeros((L,), jnp.float32)

    def body(s, carry):
        new = []
        for g in range(G):
            a0, a1 = carry[2 * g], carry[2 * g + 1]
            idx = text_v[s, pl.ds(g * L, L)]
            w = plsc.load_gather(table_v, [idx])
            x0, x1 = plsc.unpack(plsc.bitcast(w, jnp.bfloat16),
                                 format=plsc.PackFormat.INTERLEAVED)
            new.extend((a0 + x0, a1 + x1))
        return tuple(new)

    accs = plsc.parallel_loop(
        0, SEQ, 1, unroll=2, carry=(zero,) * (2 * G))(body)
    for g in range(G):
        out_v[0, pl.ds(g * L, L)] = accs[2 * g] + bias0
        out_v[1, pl.ds(g * L, L)] = accs[2 * g + 1] + bias1

    pltpu.sync_copy(out_v, out_hbm.at[:, pl.ds(base, BPW)])


@functools.cache
def _sc_reduce():
    return pl.kernel(
        _sc_body,
        out_type=jax.ShapeDtypeStruct((OUT, BATCH), jnp.float32),
        mesh=plsc.VectorSubcoreMesh(core_axis_name="c", subcore_axis_name="s",
                                    num_cores=NC, num_subcores=NS),
        compiler_params=pltpu.CompilerParams(needs_layout_passes=False),
        scratch_types=[
            pltpu.VMEM((VOCAB,), jnp.int32),
            pltpu.VMEM((SEQ, BPW), jnp.int32),
            pltpu.VMEM((OUT, BPW), jnp.float32),
            pltpu.VMEM((L,), jnp.float32),
            pltpu.SemaphoreType.DMA,
            pltpu.SemaphoreType.DMA,
        ],
    )


def kernel(text, table, W, b):
    ptable = _project_table(table.T, W)
    out_t = _sc_reduce()(ptable, text, b)
    return out_t.T


# speedup vs baseline: 68.8911x; 1.0762x over previous
import functools

import jax
import jax.numpy as jnp
from jax import lax
from jax.experimental import pallas as pl
from jax.experimental.pallas import tpu as pltpu
from jax.experimental.pallas import tpu_sc as plsc

VOCAB = 100000
EMBED = 64
OUT = 2
SEQ = 200
BATCH = 4096

NC = 2
NS = 16
L = 16
NW = NC * NS
BPW = BATCH // NW
G = BPW // L

TBLK = 16384


def _project_body(tablet_ref, w_ref, o_ref):
    t = tablet_ref[...]
    w = w_ref[...]
    inv_s = jnp.float32(1.0 / SEQ)
    p = jnp.dot(w, t, preferred_element_type=jnp.float32) * inv_s
    u = lax.bitcast_convert_type(p.astype(jnp.bfloat16), jnp.uint16)
    word = u[0, :].astype(jnp.int32) | (u[1, :].astype(jnp.int32) << 16)
    o_ref[...] = word


def _project_table(tablet, W):
    nblk = pl.cdiv(VOCAB, TBLK)
    return pl.pallas_call(
        _project_body,
        grid=(nblk,),
        in_specs=[
            pl.BlockSpec((EMBED, TBLK), lambda i: (0, i)),
            pl.BlockSpec((OUT, EMBED), lambda i: (0, 0)),
        ],
        out_specs=pl.BlockSpec((TBLK,), lambda i: (i,)),
        out_shape=jax.ShapeDtypeStruct((VOCAB,), jnp.int32),
    )(tablet, W)


def _sc_body(ptable_hbm, text_hbm, b_hbm, out_hbm,
             table_v, text_v, out_v, b_s, sem_t, sem_x):
    wid = lax.axis_index("s") * NC + lax.axis_index("c")
    base = wid * BPW

    cp_t = pltpu.make_async_copy(ptable_hbm, table_v, sem_t)
    cp_t.start()
    cp_x = pltpu.make_async_copy(
        text_hbm.at[:, pl.ds(base, BPW)], text_v, sem_x)
    cp_x.start()
    pltpu.sync_copy(b_hbm, b_s.at[pl.ds(0, OUT)])
    cp_t.wait()
    cp_x.wait()

    bv = b_s[...]
    bias0 = bv[0]
    bias1 = bv[1]
    zero = jnp.zeros((L,), jnp.float32)

    def body(s, carry):
        new = []
        for g in range(G):
            a0, a1 = carry[2 * g], carry[2 * g + 1]
            idx = text_v[s, pl.ds(g * L, L)]
            w = plsc.load_gather(table_v, [idx])
            x0, x1 = plsc.unpack(plsc.bitcast(w, jnp.bfloat16),
                                 format=plsc.PackFormat.INTERLEAVED)
            new.extend((a0 + x0, a1 + x1))
        return tuple(new)

    accs = plsc.parallel_loop(
        0, SEQ, 1, unroll=2, carry=(zero,) * (2 * G))(body)
    for g in range(G):
        out_v[0, pl.ds(g * L, L)] = accs[2 * g] + bias0
        out_v[1, pl.ds(g * L, L)] = accs[2 * g + 1] + bias1

    pltpu.sync_copy(out_v, out_hbm.at[:, pl.ds(base, BPW)])


@functools.cache
def _sc_reduce():
    return pl.kernel(
        _sc_body,
        out_type=jax.ShapeDtypeStruct((OUT, BATCH), jnp.float32),
        mesh=plsc.VectorSubcoreMesh(core_axis_name="c", subcore_axis_name="s",
                                    num_cores=NC, num_subcores=NS),
        compiler_params=pltpu.CompilerParams(needs_layout_passes=False),
        scratch_types=[
            pltpu.VMEM((VOCAB,), jnp.int32),
            pltpu.VMEM((SEQ, BPW), jnp.int32),
            pltpu.VMEM((OUT, BPW), jnp.float32),
            pltpu.VMEM((L,), jnp.float32),
            pltpu.SemaphoreType.DMA,
            pltpu.SemaphoreType.DMA,
        ],
    )


def kernel(text, table, W, b):
    ptable = _project_table(table.T, W)
    out_t = _sc_reduce()(ptable, text, b)
    return out_t.T


# speedup vs baseline: 71.1447x; 1.0327x over previous
import functools

import jax
import jax.numpy as jnp
from jax import lax
from jax.experimental import pallas as pl
from jax.experimental.pallas import tpu as pltpu
from jax.experimental.pallas import tpu_sc as plsc

VOCAB = 100000
EMBED = 64
OUT = 2
SEQ = 200
BATCH = 4096

NC = 2
NS = 16
L = 16
NW = NC * NS
BPW = BATCH // NW
G = BPW // L

TBLK = 32768
VH = 12504


def _project_body(tablet_ref, w_ref, o_ref):
    t = tablet_ref[...]
    w = w_ref[...]
    inv_s = jnp.float32(1.0 / SEQ)
    p = jnp.dot(w, t, preferred_element_type=jnp.float32) * inv_s
    u = lax.bitcast_convert_type(p.astype(jnp.bfloat16), jnp.uint16)
    word = u[0, :].astype(jnp.int32) | (u[1, :].astype(jnp.int32) << 16)
    o_ref[...] = word


def _project_table(tablet, W):
    nblk = pl.cdiv(VOCAB, TBLK)
    return pl.pallas_call(
        _project_body,
        grid=(nblk,),
        in_specs=[
            pl.BlockSpec((EMBED, TBLK), lambda i: (0, i)),
            pl.BlockSpec((OUT, EMBED), lambda i: (0, 0)),
        ],
        out_specs=pl.BlockSpec((TBLK,), lambda i: (i,)),
        out_shape=jax.ShapeDtypeStruct((VOCAB,), jnp.int32),
    )(tablet, W)


def _sc_body(ptable_hbm, text_hbm, b_hbm, out_hbm,
             table_v, text_v, out_v, b_s, table_sh, sem_t, sem_x, sem_l):
    sid = lax.axis_index("s")
    wid = sid * NC + lax.axis_index("c")
    base = wid * BPW

    cp_x = pltpu.make_async_copy(
        text_hbm.at[:, pl.ds(base, BPW)], text_v, sem_x)
    cp_x.start()
    pltpu.sync_copy(b_hbm, b_s.at[pl.ds(0, OUT)])

    cp_hi = pltpu.make_async_copy(
        ptable_hbm.at[pl.ds(VH, VOCAB - VH)],
        table_v.at[pl.ds(VH, VOCAB - VH)], sem_t)
    cp_hi.start()

    @pl.when(sid == 0)
    def _():
        pltpu.sync_copy(ptable_hbm.at[pl.ds(0, VH)], table_v.at[pl.ds(0, VH)])
        pltpu.sync_copy(table_v.at[pl.ds(0, VH)], table_sh)

    plsc.subcore_barrier()

    @pl.when(sid != 0)
    def _():
        cp_lo = pltpu.make_async_copy(
            table_sh, table_v.at[pl.ds(0, VH)], sem_l)
        cp_lo.start()
        cp_lo.wait()

    cp_x.wait()
    cp_hi.wait()

    bv = b_s[...]
    bias0 = bv[0]
    bias1 = bv[1]
    zero = jnp.zeros((L,), jnp.float32)

    def body(s, carry):
        new = []
        for g in range(G):
            a0, a1 = carry[2 * g], carry[2 * g + 1]
            idx = text_v[s, pl.ds(g * L, L)]
            w = plsc.load_gather(table_v, [idx])
            x0, x1 = plsc.unpack(plsc.bitcast(w, jnp.bfloat16),
                                 format=plsc.PackFormat.INTERLEAVED)
            new.extend((a0 + x0, a1 + x1))
        return tuple(new)

    accs = plsc.parallel_loop(
        0, SEQ, 1, unroll=2, carry=(zero,) * (2 * G))(body)
    for g in range(G):
        out_v[0, pl.ds(g * L, L)] = accs[2 * g] + bias0
        out_v[1, pl.ds(g * L, L)] = accs[2 * g + 1] + bias1

    pltpu.sync_copy(out_v, out_hbm.at[:, pl.ds(base, BPW)])


@functools.cache
def _sc_reduce():
    return pl.kernel(
        _sc_body,
        out_type=jax.ShapeDtypeStruct((OUT, BATCH), jnp.float32),
        mesh=plsc.VectorSubcoreMesh(core_axis_name="c", subcore_axis_name="s",
                                    num_cores=NC, num_subcores=NS),
        compiler_params=pltpu.CompilerParams(needs_layout_passes=False),
        scratch_types=[
            pltpu.VMEM((VOCAB,), jnp.int32),
            pltpu.VMEM((SEQ, BPW), jnp.int32),
            pltpu.VMEM((OUT, BPW), jnp.float32),
            pltpu.VMEM((L,), jnp.float32),
            pltpu.VMEM_SHARED((VH,), jnp.int32),
            pltpu.SemaphoreType.DMA,
            pltpu.SemaphoreType.DMA,
            pltpu.SemaphoreType.DMA,
        ],
    )


def kernel(text, table, W, b):
    ptable = _project_table(table.T, W)
    out_t = _sc_reduce()(ptable, text, b)
    return out_t.T
